# baseline (speedup 1.0000x reference)
import functools

import jax
import jax.numpy as jnp
from jax import lax
from jax.experimental import pallas as pl

D_MODEL = 1024
N_HEADS = 16
HEAD_DIM = D_MODEL // N_HEADS
K_SEL = 64
GS = 16
NG = 512
HI = jax.lax.Precision.HIGHEST
F32 = jnp.float32


def _means_body(x_ref, m_ref):
    ng = x_ref.shape[0] // GS
    x = x_ref[...].reshape(ng, GS, x_ref.shape[1])
    m_ref[...] = jnp.sum(x, axis=1) * (1.0 / GS)


def _score_body(m_ref, ws1_ref, bs1_ref, ws2_ref, s_ref):
    h = lax.dot_general(m_ref[...], ws1_ref[...],
                        (((1,), (1,)), ((), ()))) + bs1_ref[...]
    h = jnp.maximum(h, 0.0)
    s_ref[...] = lax.dot_general(h, ws2_ref[...], (((1,), (1,)), ((), ())))


def _core_body(m_ref, sc_ref, sr_ref, wq_ref, wk_ref, wv_ref, wo_ref,
               rk_ref, hi_ref, lo_ref):
    m = m_ref[...]
    s_col = sc_ref[...]
    s_row = sr_ref[...]
    ii = lax.broadcasted_iota(jnp.int32, (NG, NG), 0)
    jj = lax.broadcasted_iota(jnp.int32, (NG, NG), 1)
    beats = (s_col > s_row) | ((s_col == s_row) & (ii < jj))
    rank_row = jnp.sum(beats.astype(F32), axis=0, keepdims=True)
    beats_t = (s_row > s_col) | ((s_row == s_col) & (jj < ii))
    rank_col = jnp.sum(beats_t.astype(F32), axis=1, keepdims=True)
    rk_ref[...] = rank_col
    ik = lax.broadcasted_iota(jnp.int32, (K_SEL, NG), 0)
    e = (ik == rank_row.astype(jnp.int32)).astype(F32)
    sel = lax.dot_general(e, m, (((1,), (0,)), ((), ())))
    q = lax.dot_general(sel, wq_ref[...], (((1,), (1,)), ((), ())))
    k = lax.dot_general(sel, wk_ref[...], (((1,), (1,)), ((), ())))
    v = lax.dot_general(sel, wv_ref[...], (((1,), (1,)), ((), ())))
    scale = 1.0 / (HEAD_DIM ** 0.5)
    outs = []
    for hd in range(N_HEADS):
        lo = hd * HEAD_DIM
        qh = q[:, lo:lo + HEAD_DIM]
        kh = k[:, lo:lo + HEAD_DIM]
        vh = v[:, lo:lo + HEAD_DIM]
        scr = lax.dot_general(qh, kh, (((1,), (1,)), ((), ()))) * scale
        scr = scr - jnp.max(scr, axis=-1, keepdims=True)
        p = jnp.exp(scr)
        p = p / jnp.sum(p, axis=-1, keepdims=True)
        outs.append(lax.dot_general(p, vh, (((1,), (0,)), ((), ()))))
    attn = jnp.concatenate(outs, axis=1)
    attn = lax.dot_general(attn, wo_ref[...], (((1,), (1,)), ((), ())))
    attn_hi = attn.astype(jnp.bfloat16)
    hi_ref[...] = attn_hi
    lo_ref[...] = (attn - attn_hi.astype(F32)).astype(jnp.bfloat16)


def _expand_body(rk_ref, hi_ref, lo_ref, o_ref):
    bg = rk_ref.shape[0]
    d = hi_ref.shape[1]
    rk = rk_ref[...].reshape(bg, 1, 1)
    rk = jnp.broadcast_to(rk, (bg, GS, 1)).reshape(bg * GS, 1)
    ik = lax.broadcasted_iota(jnp.int32, (bg * GS, K_SEL), 1)
    eb = (ik == rk.astype(jnp.int32)).astype(jnp.bfloat16)
    o_ref[...] = (
        lax.dot_general(eb, hi_ref[...], (((1,), (0,)), ((), ())),
                        preferred_element_type=F32)
        + lax.dot_general(eb, lo_ref[...], (((1,), (0,)), ((), ())),
                          preferred_element_type=F32))


@functools.partial(jax.jit, static_argnames=("interpret",))
def kernel(hidden_states, Wq, Wk, Wv, Wo, Ws1, bs1, Ws2, bs2, interpret=False):
    B, S, D = hidden_states.shape
    ng = S // GS
    n_blk = 16
    blk_s = S // n_blk
    full = lambda *shape: pl.BlockSpec(shape, lambda *_: (0,) * len(shape))
    means = pl.pallas_call(
        _means_body,
        grid=(B, n_blk),
        in_specs=[pl.BlockSpec((None, blk_s, D), lambda b, i: (b, i, 0))],
        out_specs=pl.BlockSpec((None, blk_s // GS, D), lambda b, i: (b, i, 0)),
        out_shape=jax.ShapeDtypeStruct((B, ng, D), jnp.float32),
        interpret=interpret,
    )(hidden_states)

    means_flat = means.reshape(B * ng, D)
    s_flat = pl.pallas_call(
        _score_body,
        in_specs=[
            full(B * ng, D), full(D // 4, D), full(1, D // 4), full(1, D // 4),
        ],
        out_specs=full(B * ng, 1),
        out_shape=jax.ShapeDtypeStruct((B * ng, 1), jnp.float32),
        interpret=interpret,
    )(means_flat, Ws1, bs1.reshape(1, -1), Ws2)

    s_col = s_flat.reshape(B, ng, 1)
    s_row = s_flat.reshape(B, 1, ng)
    rank, attn_hi, attn_lo = pl.pallas_call(
        _core_body,
        grid=(B,),
        in_specs=[
            pl.BlockSpec((None, ng, D), lambda b: (b, 0, 0)),
            pl.BlockSpec((None, ng, 1), lambda b: (b, 0, 0)),
            pl.BlockSpec((None, 1, ng), lambda b: (b, 0, 0)),
            full(D, D), full(D, D), full(D, D), full(D, D),
        ],
        out_specs=[
            pl.BlockSpec((None, ng, 1), lambda b: (b, 0, 0)),
            pl.BlockSpec((None, K_SEL, D), lambda b: (b, 0, 0)),
            pl.BlockSpec((None, K_SEL, D), lambda b: (b, 0, 0)),
        ],
        out_shape=[
            jax.ShapeDtypeStruct((B, ng, 1), jnp.float32),
            jax.ShapeDtypeStruct((B, K_SEL, D), jnp.bfloat16),
            jax.ShapeDtypeStruct((B, K_SEL, D), jnp.bfloat16),
        ],
        interpret=interpret,
    )(means, s_col, s_row, Wq, Wk, Wv, Wo)

    blk_g = ng // n_blk
    out = pl.pallas_call(
        _expand_body,
        grid=(B, n_blk),
        in_specs=[
            pl.BlockSpec((None, blk_g, 1), lambda b, i: (b, i, 0)),
            pl.BlockSpec((None, K_SEL, D), lambda b, i: (b, 0, 0)),
            pl.BlockSpec((None, K_SEL, D), lambda b, i: (b, 0, 0)),
        ],
        out_specs=pl.BlockSpec((None, blk_g * GS, D), lambda b, i: (b, i, 0)),
        out_shape=jax.ShapeDtypeStruct((B, S, D), jnp.float32),
        interpret=interpret,
    )(rank, attn_hi, attn_lo)
    return out


# speedup vs baseline: 1.0127x; 1.0127x over previous
import functools

import jax
import jax.numpy as jnp
from jax import lax
from jax.experimental import pallas as pl

D_MODEL = 1024
N_HEADS = 16
HEAD_DIM = D_MODEL // N_HEADS
K_SEL = 64
GS = 16
NG = 512
F32 = jnp.float32


def _means_body(x_ref, ws1_ref, bs1_ref, ws2_ref, m_ref, s_ref):
    ng = x_ref.shape[0] // GS
    x = x_ref[...].reshape(ng, GS, x_ref.shape[1])
    m = jnp.sum(x, axis=1) * (1.0 / GS)
    m_ref[...] = m
    h = lax.dot_general(m, ws1_ref[...],
                        (((1,), (1,)), ((), ()))) + bs1_ref[...]
    h = jnp.maximum(h, 0.0)
    s_ref[...] = lax.dot_general(h, ws2_ref[...], (((1,), (1,)), ((), ())))


def _core_body(m_ref, sc_ref, sr_ref, wq_ref, wk_ref, wv_ref, wo_ref, og_ref):
    m = m_ref[...]
    s_col = sc_ref[...]
    s_row = sr_ref[...]
    ii = lax.broadcasted_iota(jnp.int32, (NG, NG), 0)
    jj = lax.broadcasted_iota(jnp.int32, (NG, NG), 1)
    beats = (s_col > s_row) | ((s_col == s_row) & (ii < jj))
    rank_row = jnp.sum(beats.astype(F32), axis=0, keepdims=True)
    ik = lax.broadcasted_iota(jnp.int32, (K_SEL, NG), 0)
    e = (ik == rank_row.astype(jnp.int32)).astype(F32)
    sel = lax.dot_general(e, m, (((1,), (0,)), ((), ())))
    q = lax.dot_general(sel, wq_ref[...], (((1,), (1,)), ((), ())))
    k = lax.dot_general(sel, wk_ref[...], (((1,), (1,)), ((), ())))
    v = lax.dot_general(sel, wv_ref[...], (((1,), (1,)), ((), ())))
    scale = 1.0 / (HEAD_DIM ** 0.5)
    outs = []
    for hd in range(N_HEADS):
        lo = hd * HEAD_DIM
        qh = q[:, lo:lo + HEAD_DIM]
        kh = k[:, lo:lo + HEAD_DIM]
        vh = v[:, lo:lo + HEAD_DIM]
        scr = lax.dot_general(qh, kh, (((1,), (1,)), ((), ()))) * scale
        scr = scr - jnp.max(scr, axis=-1, keepdims=True)
        p = jnp.exp(scr)
        p = p / jnp.sum(p, axis=-1, keepdims=True)
        outs.append(lax.dot_general(p, vh, (((1,), (0,)), ((), ()))))
    attn = jnp.concatenate(outs, axis=1)
    attn = lax.dot_general(attn, wo_ref[...], (((1,), (1,)), ((), ())))
    attn_hi = attn.astype(jnp.bfloat16).astype(F32)
    attn_lo = attn - attn_hi
    og_ref[...] = (
        lax.dot_general(e, attn_hi, (((0,), (0,)), ((), ())))
        + lax.dot_general(e, attn_lo, (((0,), (0,)), ((), ()))))


def _expand_body(og_ref, o_ref):
    bg, d = og_ref.shape
    og = og_ref[...].reshape(bg, 1, d)
    o_ref[...] = jnp.broadcast_to(og, (bg, GS, d)).reshape(bg * GS, d)


@functools.partial(jax.jit, static_argnames=("interpret",))
def kernel(hidden_states, Wq, Wk, Wv, Wo, Ws1, bs1, Ws2, bs2, interpret=False):
    B, S, D = hidden_states.shape
    ng = S // GS
    n_blk = 16
    blk_s = S // n_blk
    full = lambda *shape: pl.BlockSpec(shape, lambda *_: (0,) * len(shape))
    means, scores = pl.pallas_call(
        _means_body,
        grid=(B, n_blk),
        in_specs=[
            pl.BlockSpec((None, blk_s, D), lambda b, i: (b, i, 0)),
            full(D // 4, D), full(1, D // 4), full(1, D // 4),
        ],
        out_specs=[
            pl.BlockSpec((None, blk_s // GS, D), lambda b, i: (b, i, 0)),
            pl.BlockSpec((None, blk_s // GS, 1), lambda b, i: (b, i, 0)),
        ],
        out_shape=[
            jax.ShapeDtypeStruct((B, ng, D), jnp.float32),
            jax.ShapeDtypeStruct((B, ng, 1), jnp.float32),
        ],
        interpret=interpret,
    )(hidden_states, Ws1, bs1.reshape(1, -1), Ws2)

    s_row = scores.reshape(B, 1, ng)
    og = pl.pallas_call(
        _core_body,
        grid=(B,),
        in_specs=[
            pl.BlockSpec((None, ng, D), lambda b: (b, 0, 0)),
            pl.BlockSpec((None, ng, 1), lambda b: (b, 0, 0)),
            pl.BlockSpec((None, 1, ng), lambda b: (b, 0, 0)),
            full(D, D), full(D, D), full(D, D), full(D, D),
        ],
        out_specs=pl.BlockSpec((None, ng, D), lambda b: (b, 0, 0)),
        out_shape=jax.ShapeDtypeStruct((B, ng, D), jnp.float32),
        interpret=interpret,
    )(means, scores, s_row, Wq, Wk, Wv, Wo)

    blk_g = ng // n_blk
    out = pl.pallas_call(
        _expand_body,
        grid=(B, n_blk),
        in_specs=[pl.BlockSpec((None, blk_g, D), lambda b, i: (b, i, 0))],
        out_specs=pl.BlockSpec((None, blk_g * GS, D), lambda b, i: (b, i, 0)),
        out_shape=jax.ShapeDtypeStruct((B, S, D), jnp.float32),
        interpret=interpret,
    )(og)
    return out


# speedup vs baseline: 1.2672x; 1.2513x over previous
import functools

import jax
import jax.numpy as jnp
from jax import lax
from jax.experimental import pallas as pl

D_MODEL = 1024
N_HEADS = 16
HEAD_DIM = D_MODEL // N_HEADS
K_SEL = 64
GS = 16
NG = 512
F32 = jnp.float32


def _means_body(x_ref, ws1_ref, bs1_ref, ws2_ref, m_ref, s_ref):
    ng = x_ref.shape[0] // GS
    x = x_ref[...].reshape(ng, GS, x_ref.shape[1])
    m = jnp.sum(x, axis=1) * (1.0 / GS)
    m_ref[...] = m
    h = lax.dot_general(m, ws1_ref[...],
                        (((1,), (1,)), ((), ()))) + bs1_ref[...]
    h = jnp.maximum(h, 0.0)
    s_ref[...] = lax.dot_general(h, ws2_ref[...], (((1,), (1,)), ((), ())))


def _core_body(m_ref, sc_ref, sr_ref, wq_ref, wk_ref, wv_ref, wo_ref, og_ref):
    m = m_ref[...]
    s_col = sc_ref[...]
    s_row = sr_ref[...]
    ii = lax.broadcasted_iota(jnp.int32, (NG, NG), 0)
    jj = lax.broadcasted_iota(jnp.int32, (NG, NG), 1)
    beats = (s_col > s_row) | ((s_col == s_row) & (ii < jj))
    rank_row = jnp.sum(beats.astype(F32), axis=0, keepdims=True)
    ik = lax.broadcasted_iota(jnp.int32, (K_SEL, NG), 0)
    e = (ik == rank_row.astype(jnp.int32)).astype(F32)
    sel = lax.dot_general(e, m, (((1,), (0,)), ((), ())))
    q = lax.dot_general(sel, wq_ref[...], (((1,), (1,)), ((), ())))
    k = lax.dot_general(sel, wk_ref[...], (((1,), (1,)), ((), ())))
    v = lax.dot_general(sel, wv_ref[...], (((1,), (1,)), ((), ())))
    scale = 1.0 / (HEAD_DIM ** 0.5)
    outs = []
    for hd in range(N_HEADS):
        lo = hd * HEAD_DIM
        qh = q[:, lo:lo + HEAD_DIM]
        kh = k[:, lo:lo + HEAD_DIM]
        vh = v[:, lo:lo + HEAD_DIM]
        scr = lax.dot_general(qh, kh, (((1,), (1,)), ((), ()))) * scale
        scr = scr - jnp.max(scr, axis=-1, keepdims=True)
        p = jnp.exp(scr)
        p = p / jnp.sum(p, axis=-1, keepdims=True)
        outs.append(lax.dot_general(p, vh, (((1,), (0,)), ((), ()))))
    attn = jnp.concatenate(outs, axis=1)
    attn = lax.dot_general(attn, wo_ref[...], (((1,), (1,)), ((), ())))
    attn_hi = attn.astype(jnp.bfloat16).astype(F32)
    attn_lo = attn - attn_hi
    og_ref[...] = (
        lax.dot_general(e, attn_hi, (((0,), (0,)), ((), ())))
        + lax.dot_general(e, attn_lo, (((0,), (0,)), ((), ()))))


def _expand_body(og_ref, o_ref):
    bg, d = og_ref.shape
    og = og_ref[...].reshape(bg, 1, d)
    o_ref[...] = jnp.broadcast_to(og, (bg, GS, d)).reshape(bg * GS, d)


@functools.partial(jax.jit, static_argnames=("interpret",))
def kernel(hidden_states, Wq, Wk, Wv, Wo, Ws1, bs1, Ws2, bs2, interpret=False):
    B, S, D = hidden_states.shape
    ng = S // GS
    n_blk = 8
    blk_s = S // n_blk
    full = lambda *shape: pl.BlockSpec(shape, lambda *_: (0,) * len(shape))
    means, scores = pl.pallas_call(
        _means_body,
        grid=(B, n_blk),
        in_specs=[
            pl.BlockSpec((None, blk_s, D), lambda b, i: (b, i, 0)),
            full(D // 4, D), full(1, D // 4), full(1, D // 4),
        ],
        out_specs=[
            pl.BlockSpec((None, blk_s // GS, D), lambda b, i: (b, i, 0)),
            pl.BlockSpec((None, blk_s // GS, 1), lambda b, i: (b, i, 0)),
        ],
        out_shape=[
            jax.ShapeDtypeStruct((B, ng, D), jnp.float32),
            jax.ShapeDtypeStruct((B, ng, 1), jnp.float32),
        ],
        interpret=interpret,
    )(hidden_states, Ws1, bs1.reshape(1, -1), Ws2)

    s_row = scores.reshape(B, 1, ng)
    og = pl.pallas_call(
        _core_body,
        grid=(B,),
        in_specs=[
            pl.BlockSpec((None, ng, D), lambda b: (b, 0, 0)),
            pl.BlockSpec((None, ng, 1), lambda b: (b, 0, 0)),
            pl.BlockSpec((None, 1, ng), lambda b: (b, 0, 0)),
            full(D, D), full(D, D), full(D, D), full(D, D),
        ],
        out_specs=pl.BlockSpec((None, ng, D), lambda b: (b, 0, 0)),
        out_shape=jax.ShapeDtypeStruct((B, ng, D), jnp.float32),
        interpret=interpret,
    )(means, scores, s_row, Wq, Wk, Wv, Wo)

    blk_g = ng // n_blk
    out = pl.pallas_call(
        _expand_body,
        grid=(B, n_blk),
        in_specs=[pl.BlockSpec((None, blk_g, D), lambda b, i: (b, i, 0))],
        out_specs=pl.BlockSpec((None, blk_g * GS, D), lambda b, i: (b, i, 0)),
        out_shape=jax.ShapeDtypeStruct((B, S, D), jnp.float32),
        interpret=interpret,
    )(og)
    return out


# speedup vs baseline: 1.3144x; 1.0373x over previous
import functools

import jax
import jax.numpy as jnp
from jax import lax
from jax.experimental import pallas as pl

D_MODEL = 1024
N_HEADS = 16
HEAD_DIM = D_MODEL // N_HEADS
K_SEL = 64
GS = 16
NG = 512
F32 = jnp.float32


def _means_body(x_ref, ws1_ref, bs1_ref, ws2_ref, m_ref, s_ref):
    ng = x_ref.shape[0] // GS
    x = x_ref[...].reshape(ng, GS, x_ref.shape[1])
    m = jnp.sum(x, axis=1) * (1.0 / GS)
    m_ref[...] = m
    h = lax.dot_general(m, ws1_ref[...],
                        (((1,), (1,)), ((), ()))) + bs1_ref[...]
    h = jnp.maximum(h, 0.0)
    s_ref[...] = lax.dot_general(h, ws2_ref[...], (((1,), (1,)), ((), ())))


def _core_body(m_ref, sc_ref, sr_ref, wq_ref, wk_ref, wv_ref, wo_ref, og_ref):
    m = m_ref[...]
    s_col = sc_ref[...]
    s_row = sr_ref[...]
    ii = lax.broadcasted_iota(jnp.int32, (NG, NG), 0)
    jj = lax.broadcasted_iota(jnp.int32, (NG, NG), 1)
    beats = (s_col > s_row) | ((s_col == s_row) & (ii < jj))
    rank_row = jnp.sum(beats.astype(F32), axis=0, keepdims=True)
    ik = lax.broadcasted_iota(jnp.int32, (K_SEL, NG), 0)
    e = (ik == rank_row.astype(jnp.int32)).astype(F32)
    sel = lax.dot_general(e, m, (((1,), (0,)), ((), ())))
    q = lax.dot_general(sel, wq_ref[...], (((1,), (1,)), ((), ())))
    k = lax.dot_general(sel, wk_ref[...], (((1,), (1,)), ((), ())))
    v = lax.dot_general(sel, wv_ref[...], (((1,), (1,)), ((), ())))
    scale = 1.0 / (HEAD_DIM ** 0.5)
    outs = []
    for hd in range(N_HEADS):
        lo = hd * HEAD_DIM
        qh = q[:, lo:lo + HEAD_DIM]
        kh = k[:, lo:lo + HEAD_DIM]
        vh = v[:, lo:lo + HEAD_DIM]
        scr = lax.dot_general(qh, kh, (((1,), (1,)), ((), ()))) * scale
        scr = scr - jnp.max(scr, axis=-1, keepdims=True)
        p = jnp.exp(scr)
        p = p / jnp.sum(p, axis=-1, keepdims=True)
        outs.append(lax.dot_general(p, vh, (((1,), (0,)), ((), ()))))
    attn = jnp.concatenate(outs, axis=1)
    attn = lax.dot_general(attn, wo_ref[...], (((1,), (1,)), ((), ())))
    attn_hi = attn.astype(jnp.bfloat16).astype(F32)
    attn_lo = attn - attn_hi
    og_ref[...] = (
        lax.dot_general(e, attn_hi, (((0,), (0,)), ((), ())))
        + lax.dot_general(e, attn_lo, (((0,), (0,)), ((), ()))))


def _expand_body(og_ref, o_ref):
    bg, d = og_ref.shape
    og = og_ref[...].reshape(bg, 1, d)
    o_ref[...] = jnp.broadcast_to(og, (bg, GS, d)).reshape(bg * GS, d)


@functools.partial(jax.jit, static_argnames=("interpret",))
def kernel(hidden_states, Wq, Wk, Wv, Wo, Ws1, bs1, Ws2, bs2, interpret=False):
    B, S, D = hidden_states.shape
    ng = S // GS
    n_blk = 4
    blk_s = S // n_blk
    full = lambda *shape: pl.BlockSpec(shape, lambda *_: (0,) * len(shape))
    means, scores = pl.pallas_call(
        _means_body,
        grid=(B, n_blk),
        in_specs=[
            pl.BlockSpec((None, blk_s, D), lambda b, i: (b, i, 0)),
            full(D // 4, D), full(1, D // 4), full(1, D // 4),
        ],
        out_specs=[
            pl.BlockSpec((None, blk_s // GS, D), lambda b, i: (b, i, 0)),
            pl.BlockSpec((None, blk_s // GS, 1), lambda b, i: (b, i, 0)),
        ],
        out_shape=[
            jax.ShapeDtypeStruct((B, ng, D), jnp.float32),
            jax.ShapeDtypeStruct((B, ng, 1), jnp.float32),
        ],
        interpret=interpret,
    )(hidden_states, Ws1, bs1.reshape(1, -1), Ws2)

    s_row = scores.reshape(B, 1, ng)
    og = pl.pallas_call(
        _core_body,
        grid=(B,),
        in_specs=[
            pl.BlockSpec((None, ng, D), lambda b: (b, 0, 0)),
            pl.BlockSpec((None, ng, 1), lambda b: (b, 0, 0)),
            pl.BlockSpec((None, 1, ng), lambda b: (b, 0, 0)),
            full(D, D), full(D, D), full(D, D), full(D, D),
        ],
        out_specs=pl.BlockSpec((None, ng, D), lambda b: (b, 0, 0)),
        out_shape=jax.ShapeDtypeStruct((B, ng, D), jnp.float32),
        interpret=interpret,
    )(means, scores, s_row, Wq, Wk, Wv, Wo)

    blk_g = ng // n_blk
    out = pl.pallas_call(
        _expand_body,
        grid=(B, n_blk),
        in_specs=[pl.BlockSpec((None, blk_g, D), lambda b, i: (b, i, 0))],
        out_specs=pl.BlockSpec((None, blk_g * GS, D), lambda b, i: (b, i, 0)),
        out_shape=jax.ShapeDtypeStruct((B, S, D), jnp.float32),
        interpret=interpret,
    )(og)
    return out
